# baseline (speedup 1.0000x reference)
import functools

import jax
import jax.numpy as jnp
from jax import lax
from jax.experimental import pallas as pl
from jax.experimental.pallas import tpu as pltpu
from jax.experimental.pallas import tpu_sc as plsc

_NUM_CORES = 2
_NUM_SUBCORES = 16
_NW = _NUM_CORES * _NUM_SUBCORES
_NBUF = 8
_GD = 4


@functools.lru_cache(maxsize=None)
def _make_gather(V, D, Bt, H, Hp, Dp):
    rows_per_w = Bt // _NW
    idx_rows = rows_per_w // 2
    mesh = plsc.VectorSubcoreMesh(core_axis_name="c", subcore_axis_name="s")

    @functools.partial(
        pl.kernel,
        out_type=jax.ShapeDtypeStruct((Bt, Hp, Dp), jnp.float32),
        mesh=mesh,
        scratch_types=[
            pltpu.VMEM((idx_rows, 128), jnp.int32),
            pltpu.VMEM((_NBUF, Hp, D), jnp.float32),
            pltpu.SemaphoreType.DMA,
            pltpu.SemaphoreType.DMA,
        ],
        compiler_params=pltpu.CompilerParams(use_tc_tiling_on_sc=False),
    )
    def k(table_hbm, idx_hbm, out_hbm, idx_v, rows_v, gsem, wsem):
        wid = lax.axis_index("s") * _NUM_CORES + lax.axis_index("c")
        rbase = wid * rows_per_w
        pltpu.sync_copy(idx_hbm.at[wid], idx_v)

        def g_start(row, off, s):
            pltpu.async_copy(
                table_hbm.at[idx_v.at[row, pl.ds(off, Hp)]], rows_v.at[s], gsem)

        def g_wait(row, off, s):
            pltpu.make_async_copy(
                table_hbm.at[idx_v.at[row, pl.ds(off, Hp)]], rows_v.at[s], gsem).wait()

        def w_start(m, s):
            pltpu.async_copy(
                rows_v.at[s],
                out_hbm.at[rbase + m, pl.ds(0, Hp), pl.ds(0, D)], wsem)

        def w_wait(m, s):
            pltpu.make_async_copy(
                rows_v.at[s],
                out_hbm.at[rbase + m, pl.ds(0, Hp), pl.ds(0, D)], wsem).wait()

        def gs(m_base, b, s):
            g_start(m_base // 2 + b // 2, (b % 2) * 64, s)

        def gw(m_base, b, s):
            g_wait(m_base // 2 + b // 2, (b % 2) * 64, s)

        for m in range(_GD):
            gs(0, m, m)
        for m in range(_NBUF - _GD):
            gw(0, m, m % _NBUF)
            w_start(m, m % _NBUF)
            gs(_GD, m, (m + _GD) % _NBUF)

        n_main = (rows_per_w - _NBUF) // _NBUF

        def body(g, carry):
            m0 = (_NBUF - _GD) + g * _NBUF
            for b in range(_NBUF):
                m = m0 + b
                s = (_NBUF - _GD + b) % _NBUF
                sn = b
                gw(m0, b, s)
                w_start(m, s)
                w_wait(m + _GD - _NBUF, sn)
                gs(m0 + _GD, b, sn)
            return carry

        lax.fori_loop(0, n_main, body, 0)

        for i in range(_GD):
            m = rows_per_w - _GD + i
            gw(rows_per_w - _GD, i, m % _NBUF)
            w_start(m, m % _NBUF)
        for i in range(_NBUF):
            m = rows_per_w - _NBUF + i
            w_wait(m, m % _NBUF)

    return k


def kernel(token_ids, embedding):
    Bt, H = token_ids.shape
    V, D = embedding.shape
    Hp = (H + 7) // 8 * 8
    Dp = 128
    idx = jnp.pad(
        token_ids.astype(jnp.int32).reshape(_NW, Bt // _NW // 2, 2, H),
        ((0, 0), (0, 0), (0, 0), (0, 64 - H)),
    ).reshape(_NW, Bt // _NW // 2, 128)
    out = _make_gather(V, D, Bt, H, Hp, Dp)(embedding, idx)
    return out[:, :H, :D]


# speedup vs baseline: 3.3459x; 3.3459x over previous
import functools

import jax
import jax.numpy as jnp
from jax import lax
from jax.experimental import pallas as pl
from jax.experimental.pallas import tpu as pltpu
from jax.experimental.pallas import tpu_sc as plsc

_NUM_CORES = 2
_NUM_SUBCORES = 16
_NW = _NUM_CORES * _NUM_SUBCORES
_NBUF = 16
_GD = 8


@functools.lru_cache(maxsize=None)
def _make_gather(V, D, Bt, H, Hp, Dp):
    rows_per_w = Bt // _NW
    mesh = plsc.VectorSubcoreMesh(core_axis_name="c", subcore_axis_name="s")

    @functools.partial(
        pl.kernel,
        out_type=jax.ShapeDtypeStruct((Bt, Hp, Dp), jnp.float32),
        mesh=mesh,
        scratch_types=[
            pltpu.VMEM((rows_per_w, H), jnp.int32),
            pltpu.VMEM((_NBUF, H, D), jnp.float32),
            pltpu.SemaphoreType.DMA,
            pltpu.SemaphoreType.DMA,
        ],
        compiler_params=pltpu.CompilerParams(use_tc_tiling_on_sc=False),
    )
    def k(table_hbm, idx_hbm, out_hbm, idx_v, rows_v, gsem, wsem):
        wid = lax.axis_index("s") * _NUM_CORES + lax.axis_index("c")
        rbase = wid * rows_per_w
        pltpu.sync_copy(idx_hbm.at[wid], idx_v)

        def g_start(m, s):
            pltpu.async_copy(table_hbm.at[idx_v.at[m]], rows_v.at[s], gsem)

        def g_wait(m, s):
            pltpu.make_async_copy(table_hbm.at[idx_v.at[m]], rows_v.at[s], gsem).wait()

        def w_start(m, s):
            pltpu.async_copy(
                rows_v.at[s],
                out_hbm.at[rbase + m, pl.ds(0, H), pl.ds(0, D)], wsem)

        def w_wait(m, s):
            pltpu.make_async_copy(
                rows_v.at[s],
                out_hbm.at[rbase + m, pl.ds(0, H), pl.ds(0, D)], wsem).wait()

        for m in range(_GD):
            g_start(m, m)
        for m in range(_NBUF - _GD):
            g_wait(m, m % _NBUF)
            w_start(m, m % _NBUF)
            g_start(m + _GD, (m + _GD) % _NBUF)

        n_main = (rows_per_w - _NBUF) // _NBUF

        def body(g, carry):
            m0 = (_NBUF - _GD) + g * _NBUF
            for b in range(_NBUF):
                m = m0 + b
                s = (_NBUF - _GD + b) % _NBUF
                sn = b
                g_wait(m, s)
                w_start(m, s)
                w_wait(m + _GD - _NBUF, sn)
                g_start(m + _GD, sn)
            return carry

        lax.fori_loop(0, n_main, body, 0)

        for i in range(_GD):
            m = rows_per_w - _GD + i
            g_wait(m, m % _NBUF)
            w_start(m, m % _NBUF)
        for i in range(_NBUF):
            m = rows_per_w - _NBUF + i
            w_wait(m, m % _NBUF)

    return k


def kernel(token_ids, embedding):
    Bt, H = token_ids.shape
    V, D = embedding.shape
    Hp = (H + 7) // 8 * 8
    Dp = 128
    idx = token_ids.reshape(_NW, Bt // _NW, H).astype(jnp.int32)
    out = _make_gather(V, D, Bt, H, Hp, Dp)(embedding, idx)
    return out[:, :H, :D]


# speedup vs baseline: 3.3498x; 1.0012x over previous
import functools

import jax
import jax.numpy as jnp
from jax import lax
from jax.experimental import pallas as pl
from jax.experimental.pallas import tpu as pltpu
from jax.experimental.pallas import tpu_sc as plsc

_NUM_CORES = 2
_NUM_SUBCORES = 16
_NW = _NUM_CORES * _NUM_SUBCORES
_NBUF = 16
_GD = 8


@functools.lru_cache(maxsize=None)
def _make_gather(V, D, Bt, H, Hp, Dp):
    rows_per_w = Bt // _NW
    mesh = plsc.VectorSubcoreMesh(core_axis_name="c", subcore_axis_name="s")

    @functools.partial(
        pl.kernel,
        out_type=jax.ShapeDtypeStruct((Bt, Hp, Dp), jnp.float32),
        mesh=mesh,
        scratch_types=[
            pltpu.VMEM((rows_per_w, H), jnp.int32),
            pltpu.VMEM((_NBUF, H, D), jnp.float32),
            pltpu.SemaphoreType.DMA,
            pltpu.SemaphoreType.DMA,
        ],
        compiler_params=pltpu.CompilerParams(use_tc_tiling_on_sc=False),
    )
    def k(table_hbm, idx_hbm, out_hbm, idx_v, rows_v, gsem, wsem):
        wid = lax.axis_index("s") * _NUM_CORES + lax.axis_index("c")
        rbase = wid * rows_per_w
        pltpu.sync_copy(idx_hbm.at[pl.ds(rbase, rows_per_w)], idx_v)

        def g_start(m, s):
            pltpu.async_copy(table_hbm.at[idx_v.at[m]], rows_v.at[s], gsem)

        def g_wait(m, s):
            pltpu.make_async_copy(table_hbm.at[idx_v.at[m]], rows_v.at[s], gsem).wait()

        def w_start(m, s):
            pltpu.async_copy(
                rows_v.at[s],
                out_hbm.at[rbase + m, pl.ds(0, H), pl.ds(0, D)], wsem)

        def w_wait(m, s):
            pltpu.make_async_copy(
                rows_v.at[s],
                out_hbm.at[rbase + m, pl.ds(0, H), pl.ds(0, D)], wsem).wait()

        for m in range(_GD):
            g_start(m, m)
        for m in range(_NBUF - _GD):
            g_wait(m, m % _NBUF)
            w_start(m, m % _NBUF)
            g_start(m + _GD, (m + _GD) % _NBUF)

        n_main = (rows_per_w - _NBUF) // _NBUF

        def body(g, carry):
            m0 = (_NBUF - _GD) + g * _NBUF
            for b in range(_NBUF):
                m = m0 + b
                s = (_NBUF - _GD + b) % _NBUF
                sn = b
                g_wait(m, s)
                w_start(m, s)
                w_wait(m + _GD - _NBUF, sn)
                g_start(m + _GD, sn)
            return carry

        lax.fori_loop(0, n_main, body, 0)

        for i in range(_GD):
            m = rows_per_w - _GD + i
            g_wait(m, m % _NBUF)
            w_start(m, m % _NBUF)
        for i in range(_NBUF):
            m = rows_per_w - _NBUF + i
            w_wait(m, m % _NBUF)

    return k


def kernel(token_ids, embedding):
    Bt, H = token_ids.shape
    V, D = embedding.shape
    Hp = (H + 7) // 8 * 8
    Dp = 128
    out = _make_gather(V, D, Bt, H, Hp, Dp)(embedding, token_ids.astype(jnp.int32))
    return out[:, :H, :D]


# speedup vs baseline: 3.3729x; 1.0069x over previous
import functools

import jax
import jax.numpy as jnp
from jax import lax
from jax.experimental import pallas as pl
from jax.experimental.pallas import tpu as pltpu
from jax.experimental.pallas import tpu_sc as plsc

_NUM_CORES = 2
_NUM_SUBCORES = 16
_NW = _NUM_CORES * _NUM_SUBCORES
_NBUF = 16
_GD = 12


@functools.lru_cache(maxsize=None)
def _make_gather(V, D, Bt, H, Hp, Dp):
    rows_per_w = Bt // _NW
    mesh = plsc.VectorSubcoreMesh(core_axis_name="c", subcore_axis_name="s")

    @functools.partial(
        pl.kernel,
        out_type=jax.ShapeDtypeStruct((Bt, Hp, Dp), jnp.float32),
        mesh=mesh,
        scratch_types=[
            pltpu.VMEM((rows_per_w, H), jnp.int32),
            pltpu.VMEM((_NBUF, H, D), jnp.float32),
            pltpu.SemaphoreType.DMA,
            pltpu.SemaphoreType.DMA,
        ],
        compiler_params=pltpu.CompilerParams(use_tc_tiling_on_sc=False),
    )
    def k(table_hbm, idx_hbm, out_hbm, idx_v, rows_v, gsem, wsem):
        wid = lax.axis_index("s") * _NUM_CORES + lax.axis_index("c")
        rbase = wid * rows_per_w
        pltpu.sync_copy(idx_hbm.at[pl.ds(rbase, rows_per_w)], idx_v)

        def g_start(m, s):
            pltpu.async_copy(table_hbm.at[idx_v.at[m]], rows_v.at[s], gsem)

        def g_wait(m, s):
            pltpu.make_async_copy(table_hbm.at[idx_v.at[m]], rows_v.at[s], gsem).wait()

        def w_start(m, s):
            pltpu.async_copy(
                rows_v.at[s],
                out_hbm.at[rbase + m, pl.ds(0, H), pl.ds(0, D)], wsem)

        def w_wait(m, s):
            pltpu.make_async_copy(
                rows_v.at[s],
                out_hbm.at[rbase + m, pl.ds(0, H), pl.ds(0, D)], wsem).wait()

        for m in range(_GD):
            g_start(m, m)
        for m in range(_NBUF - _GD):
            g_wait(m, m % _NBUF)
            w_start(m, m % _NBUF)
            g_start(m + _GD, (m + _GD) % _NBUF)

        n_main = (rows_per_w - _NBUF) // _NBUF

        def body(g, carry):
            m0 = (_NBUF - _GD) + g * _NBUF
            for b in range(_NBUF):
                m = m0 + b
                s = (_NBUF - _GD + b) % _NBUF
                sn = b
                g_wait(m, s)
                w_start(m, s)
                w_wait(m + _GD - _NBUF, sn)
                g_start(m + _GD, sn)
            return carry

        lax.fori_loop(0, n_main, body, 0)

        for i in range(_GD):
            m = rows_per_w - _GD + i
            g_wait(m, m % _NBUF)
            w_start(m, m % _NBUF)
        for i in range(_NBUF):
            m = rows_per_w - _NBUF + i
            w_wait(m, m % _NBUF)

    return k


def kernel(token_ids, embedding):
    Bt, H = token_ids.shape
    V, D = embedding.shape
    Hp = (H + 7) // 8 * 8
    Dp = 128
    out = _make_gather(V, D, Bt, H, Hp, Dp)(embedding, token_ids.astype(jnp.int32))
    return out[:, :H, :D]
